# baseline (speedup 1.0000x reference)
import functools

import jax
import jax.numpy as jnp
from jax.experimental import pallas as pl
from jax.experimental.pallas import tpu as pltpu

_EPS = 1e-6
_CAP_FACTOR = 1.25
_NBUF = 3
_CHUNKS = (256, 1024, 1024, 1024, 512, 256)


def _masked_softmax(logits, e):
    m = jnp.max(logits, axis=0, keepdims=True)
    ex = jnp.exp(logits - m)
    probs = ex / jnp.sum(ex, axis=0, keepdims=True)

    iota = jax.lax.broadcasted_iota(jnp.int32, logits.shape, 0)
    i1 = jnp.min(jnp.where(logits == m, iota, e), axis=0, keepdims=True)
    mask1 = iota == i1
    l2 = jnp.where(mask1, jnp.float32(-jnp.inf), logits)
    m2 = jnp.max(l2, axis=0, keepdims=True)
    i2 = jnp.min(jnp.where(l2 == m2, iota, e), axis=0, keepdims=True)
    mask = mask1 | (iota == i2)
    return jnp.where(mask, probs, jnp.float32(0.0))


def _gate_kernel(x_ref, w_ref, b_ref, o_ref, buf, sems, *, capacity, batch):
    e = w_ref.shape[0]
    starts = []
    s0 = 0
    for sz in _CHUNKS:
        starts.append(s0)
        s0 += sz
    work = [(s, sz, b) for s, sz in zip(starts, _CHUNKS) for b in range(batch)]

    def copy(k, slot):
        s, sz, b = work[k]
        return pltpu.make_async_copy(
            x_ref.at[b, s:s + sz, :], buf.at[slot, 0:sz, :], sems.at[slot])

    for k in range(_NBUF):
        copy(k, k).start()

    bias = jnp.transpose(b_ref[...], (1, 0))
    w = w_ref[...]
    for k, (s, sz, b) in enumerate(work):
        slot = k % _NBUF
        copy(k, slot).wait()
        x = buf[slot, 0:sz, :]
        logits = jax.lax.dot_general(
            w, x, (((1,), (1,)), ((), ())),
            preferred_element_type=jnp.float32)
        masked = _masked_softmax(logits + bias, e)
        if k + _NBUF < len(work):
            copy(k + _NBUF, slot).start()
        if b < batch - 1:
            o_ref[b, :, s:s + sz] = masked
        else:
            prev = [o_ref[bi, :, s:s + sz] for bi in range(batch - 1)]
            den = masked + jnp.float32(_EPS)
            for p in prev:
                den = den + p
            scale = jnp.float32(capacity) / den
            for bi in range(batch - 1):
                o_ref[bi, :, s:s + sz] = prev[bi] * scale
            o_ref[batch - 1, :, s:s + sz] = masked * scale


def kernel(X, Wg, bg):
    batch, seq, dim = X.shape
    e = Wg.shape[0]
    capacity = int(_CAP_FACTOR * batch)
    maxchunk = max(_CHUNKS)
    out = pl.pallas_call(
        functools.partial(_gate_kernel, capacity=capacity, batch=batch),
        in_specs=[
            pl.BlockSpec(memory_space=pl.ANY),
            pl.BlockSpec(memory_space=pltpu.MemorySpace.VMEM),
            pl.BlockSpec(memory_space=pltpu.MemorySpace.VMEM),
        ],
        out_specs=pl.BlockSpec(memory_space=pltpu.MemorySpace.VMEM),
        out_shape=jax.ShapeDtypeStruct((batch, e, seq), jnp.float32),
        scratch_shapes=[
            pltpu.VMEM((_NBUF, maxchunk, dim), jnp.float32),
            pltpu.SemaphoreType.DMA((_NBUF,)),
        ],
    )(X, Wg, bg.reshape(1, e))
    return (jnp.transpose(out, (0, 2, 1)), None)


# speedup vs baseline: 1.0335x; 1.0335x over previous
import functools

import jax
import jax.numpy as jnp
from jax.experimental import pallas as pl

_EPS = 1e-6
_CAP_FACTOR = 1.25


def _masked_softmax(logits, e):
    m = jnp.max(logits, axis=0, keepdims=True)
    ex = jnp.exp(logits - m)
    probs = ex / jnp.sum(ex, axis=0, keepdims=True)

    iota = jax.lax.broadcasted_iota(jnp.int32, logits.shape, 0)
    i1 = jnp.min(jnp.where(logits == m, iota, e), axis=0, keepdims=True)
    mask1 = iota == i1
    l2 = jnp.where(mask1, jnp.float32(-jnp.inf), logits)
    m2 = jnp.max(l2, axis=0, keepdims=True)
    i2 = jnp.min(jnp.where(l2 == m2, iota, e), axis=0, keepdims=True)
    mask = mask1 | (iota == i2)
    return jnp.where(mask, probs, jnp.float32(0.0))


def _gate_kernel(x_ref, w_ref, b_ref, o_ref, *, capacity, batch):
    _, sblk, dim = x_ref.shape
    e = w_ref.shape[0]
    b = pl.program_id(1)
    x = x_ref[...].reshape(sblk, dim)
    logits = jax.lax.dot_general(
        w_ref[...], x, (((1,), (1,)), ((), ())),
        preferred_element_type=jnp.float32)
    logits = logits + jnp.transpose(b_ref[...], (1, 0))
    masked = _masked_softmax(logits, e)

    for bi in range(batch - 1):
        @pl.when(b == bi)
        def _(bi=bi):
            o_ref[bi] = masked

    @pl.when(b == batch - 1)
    def _():
        prev = [o_ref[bi] for bi in range(batch - 1)]
        den = masked + jnp.float32(_EPS)
        for p in prev:
            den = den + p
        scale = jnp.float32(capacity) / den
        for bi in range(batch - 1):
            o_ref[bi] = prev[bi] * scale
        o_ref[batch - 1] = masked * scale


def kernel(X, Wg, bg):
    batch, seq, dim = X.shape
    e = Wg.shape[0]
    capacity = int(_CAP_FACTOR * batch)
    sblk = 1024
    grid = (seq // sblk, batch)
    out = pl.pallas_call(
        functools.partial(_gate_kernel, capacity=capacity, batch=batch),
        grid=grid,
        in_specs=[
            pl.BlockSpec((1, sblk, dim), lambda i, b: (b, i, 0)),
            pl.BlockSpec((e, dim), lambda i, b: (0, 0)),
            pl.BlockSpec((1, e), lambda i, b: (0, 0)),
        ],
        out_specs=pl.BlockSpec((batch, e, sblk), lambda i, b: (0, 0, i)),
        out_shape=jax.ShapeDtypeStruct((batch, e, seq), jnp.float32),
    )(X, Wg, bg.reshape(1, e))
    return (jnp.transpose(out, (0, 2, 1)), None)
